# Initial kernel scaffold; baseline (speedup 1.0000x reference)
import jax
import jax.numpy as jnp
from jax.experimental import pallas as pl


def kernel(features, edge_index, W_self_0, W_neigh_0, b_0, W_self_1, W_neigh_1, b_1, W_self_2, W_neigh_2, b_2):
    raise NotImplementedError("write your pallas kernel here")


# speedup vs baseline: 4.8614x; 4.8614x over previous
import functools

import jax
import jax.numpy as jnp
from jax import lax
from jax.experimental import pallas as pl
from jax.experimental.pallas import tpu as pltpu
from jax.experimental.pallas import tpu_sc as plsc

N = 10000
E = 320000
IN = 128
HID = 128
CLS = 40

NC = 2
NS = 16
NW = NC * NS
EW = E // NW
CHUNK = 80
NCHUNK = EW // CHUNK
RPT = N // NS
ZROWS = 25


def _fill(ref, rows, width, value):
    vec = jnp.full((16,), value, dtype=jnp.float32)
    per_row = width // 16

    def body(t, _):
        r = t // per_row
        col = (t % per_row) * 16
        ref[r, pl.ds(col, 16)] = vec
        return _

    lax.fori_loop(0, rows * per_row, body, None)


def _make_sc_agg(feat_width, with_deg, interpret=False):
    mesh = plsc.VectorSubcoreMesh(core_axis_name="c", subcore_axis_name="s",
                                  num_cores=NC, num_subcores=NS)
    agg_type = jax.ShapeDtypeStruct((NC * N, feat_width), jnp.float32)
    out_type = ([agg_type, jax.ShapeDtypeStruct((NC * N, 16), jnp.float32)]
                if with_deg else agg_type)
    scratch = [
        pltpu.VMEM((CHUNK,), jnp.int32),
        pltpu.VMEM((CHUNK,), jnp.int32),
        pltpu.VMEM((CHUNK, feat_width), jnp.float32),
        pltpu.VMEM((ZROWS, feat_width), jnp.float32),
        pltpu.VMEM_SHARED((N, feat_width), jnp.float32),
        pltpu.SemaphoreType.DMA,
    ]
    if with_deg:
        scratch += [
            pltpu.VMEM((CHUNK, 16), jnp.float32),
            pltpu.VMEM((ZROWS, 16), jnp.float32),
            pltpu.VMEM_SHARED((N, 16), jnp.float32),
        ]

    @functools.partial(
        pl.kernel,
        out_type=out_type,
        mesh=mesh,
        scratch_types=scratch,
        compiler_params=pltpu.CompilerParams(use_tc_tiling_on_sc=False),
        interpret=interpret,
    )
    def sc_agg(p_hbm, src_hbm, dst_hbm, *refs):
        if with_deg:
            (out_hbm, deg_hbm, src_v, dst_v, rows_v, zbuf, acc, sem,
             ones_v, zbuf16, dacc) = refs
        else:
            out_hbm, src_v, dst_v, rows_v, zbuf, acc, sem = refs
        c = lax.axis_index("c")
        s = lax.axis_index("s")
        w = c * NS + s
        ebase = w * EW
        rbase = s * RPT

        _fill(zbuf, ZROWS, feat_width, 0.0)
        if with_deg:
            _fill(zbuf16, ZROWS, 16, 0.0)
            _fill(ones_v, CHUNK, 16, 1.0)

        def zero_body(i, _):
            pltpu.sync_copy(zbuf, acc.at[pl.ds(rbase + i * ZROWS, ZROWS)])
            if with_deg:
                pltpu.sync_copy(zbuf16, dacc.at[pl.ds(rbase + i * ZROWS, ZROWS)])
            return _

        lax.fori_loop(0, RPT // ZROWS, zero_body, None)
        plsc.subcore_barrier()

        def body(j, _):
            off = ebase + j * CHUNK
            pltpu.sync_copy(src_hbm.at[pl.ds(off, CHUNK)], src_v)
            pltpu.sync_copy(dst_hbm.at[pl.ds(off, CHUNK)], dst_v)
            pltpu.async_copy(p_hbm.at[src_v], rows_v, sem).wait()
            pltpu.sync_copy(rows_v, acc.at[dst_v], add=True)
            if with_deg:
                pltpu.sync_copy(ones_v, dacc.at[dst_v], add=True)
            return _

        lax.fori_loop(0, NCHUNK, body, None)
        plsc.subcore_barrier()

        obase = c * N + rbase
        pltpu.sync_copy(acc.at[pl.ds(rbase, RPT)], out_hbm.at[pl.ds(obase, RPT)])
        if with_deg:
            pltpu.sync_copy(dacc.at[pl.ds(rbase, RPT)],
                            deg_hbm.at[pl.ds(obase, RPT)])

    return sc_agg


_make_sc_agg = functools.lru_cache(maxsize=None)(_make_sc_agg)

BN = 2000


def _row_spec(width):
    return pl.BlockSpec((BN, width), lambda i: (i, 0))


def _full_spec(shape):
    return pl.BlockSpec(shape, lambda i: tuple(0 for _ in shape))


def _mm_first_body(x_ref, wn_ref, ws_ref, b_ref, p_ref, s_ref):
    x = x_ref[...]
    p_ref[...] = jnp.dot(x, wn_ref[...], preferred_element_type=jnp.float32)
    s_ref[...] = (jnp.dot(x, ws_ref[...], preferred_element_type=jnp.float32)
                  + b_ref[...])


def _mm_first(x, wn, ws, b, interpret=False):
    return pl.pallas_call(
        _mm_first_body,
        grid=(N // BN,),
        in_specs=[_row_spec(IN), _full_spec((IN, HID)), _full_spec((IN, HID)),
                  _full_spec((1, HID))],
        out_specs=[_row_spec(HID), _row_spec(HID)],
        out_shape=[jax.ShapeDtypeStruct((N, HID), jnp.float32),
                   jax.ShapeDtypeStruct((N, HID), jnp.float32)],
        interpret=interpret,
    )(x, wn, ws, b)


def _mm_mid_body(sp_ref, a0_ref, a1_ref, d0_ref, d1_ref, wn_ref, ws_ref,
                 b_ref, p_ref, s_ref, inv_ref):
    deg = d0_ref[...][:, :1] + d1_ref[...][:, :1]
    inv = 1.0 / jnp.maximum(deg, 1.0)
    h = jnp.maximum(sp_ref[...] + (a0_ref[...] + a1_ref[...]) * inv, 0.0)
    p_ref[...] = jnp.dot(h, wn_ref[...], preferred_element_type=jnp.float32)
    s_ref[...] = (jnp.dot(h, ws_ref[...], preferred_element_type=jnp.float32)
                  + b_ref[...])
    inv_ref[...] = jnp.broadcast_to(inv, (BN, 16))


def _mm_mid(s_prev, a0, a1, d0, d1, wn, ws, b, interpret=False):
    return pl.pallas_call(
        _mm_mid_body,
        grid=(N // BN,),
        in_specs=[_row_spec(HID), _row_spec(HID), _row_spec(HID),
                  _row_spec(16), _row_spec(16),
                  _full_spec((HID, HID)), _full_spec((HID, HID)),
                  _full_spec((1, HID))],
        out_specs=[_row_spec(HID), _row_spec(HID), _row_spec(16)],
        out_shape=[jax.ShapeDtypeStruct((N, HID), jnp.float32),
                   jax.ShapeDtypeStruct((N, HID), jnp.float32),
                   jax.ShapeDtypeStruct((N, 16), jnp.float32)],
        interpret=interpret,
    )(s_prev, a0, a1, d0, d1, wn, ws, b)


def _mm_last_body(sp_ref, a0_ref, a1_ref, inv_ref, wn_ref, ws_ref, b_ref,
                  p_ref, s_ref):
    inv = inv_ref[...][:, :1]
    h = jnp.maximum(sp_ref[...] + (a0_ref[...] + a1_ref[...]) * inv, 0.0)
    p_ref[...] = jnp.dot(h, wn_ref[...], preferred_element_type=jnp.float32)
    s_ref[...] = (jnp.dot(h, ws_ref[...], preferred_element_type=jnp.float32)
                  + b_ref[...])


def _mm_last(s_prev, a0, a1, inv, wn_pad, ws, b, interpret=False):
    return pl.pallas_call(
        _mm_last_body,
        grid=(N // BN,),
        in_specs=[_row_spec(HID), _row_spec(HID), _row_spec(HID),
                  _row_spec(16),
                  _full_spec((HID, 64)), _full_spec((HID, CLS)),
                  _full_spec((1, CLS))],
        out_specs=[_row_spec(64), _row_spec(CLS)],
        out_shape=[jax.ShapeDtypeStruct((N, 64), jnp.float32),
                   jax.ShapeDtypeStruct((N, CLS), jnp.float32)],
        interpret=interpret,
    )(s_prev, a0, a1, inv, wn_pad, ws, b)


def _final_body(s_ref, a0_ref, a1_ref, inv_ref, o_ref):
    agg = a0_ref[...][:, :CLS] + a1_ref[...][:, :CLS]
    o_ref[...] = s_ref[...] + agg * inv_ref[...][:, :1]


def _final(s2, a0, a1, inv, interpret=False):
    return pl.pallas_call(
        _final_body,
        grid=(N // BN,),
        in_specs=[_row_spec(CLS), _row_spec(64), _row_spec(64),
                  _row_spec(16)],
        out_specs=_row_spec(CLS),
        out_shape=jax.ShapeDtypeStruct((N, CLS), jnp.float32),
        interpret=interpret,
    )(s2, a0, a1, inv)


def kernel(features, edge_index, W_self_0, W_neigh_0, b_0, W_self_1,
           W_neigh_1, b_1, W_self_2, W_neigh_2, b_2):
    src = edge_index[0]
    dst = edge_index[1]

    p0, s0 = _mm_first(features, W_neigh_0, W_self_0, b_0.reshape(1, HID))
    agg0, deg = _make_sc_agg(HID, with_deg=True)(p0, src, dst)
    a0, a1 = agg0[:N], agg0[N:]
    d0, d1 = deg[:N], deg[N:]

    p1, s1, inv = _mm_mid(s0, a0, a1, d0, d1, W_neigh_1, W_self_1,
                          b_1.reshape(1, HID))
    agg1 = _make_sc_agg(HID, with_deg=False)(p1, src, dst)

    wn2_pad = jnp.pad(W_neigh_2, ((0, 0), (0, 64 - CLS)))
    p2, s2 = _mm_last(s1, agg1[:N], agg1[N:], inv, wn2_pad, W_self_2,
                      b_2.reshape(1, CLS))
    agg2 = _make_sc_agg(64, with_deg=False)(p2, src, dst)

    return _final(s2, agg2[:N], agg2[N:], inv)


# speedup vs baseline: 10.7287x; 2.2069x over previous
import functools

import jax
import jax.numpy as jnp
from jax import lax
from jax.experimental import pallas as pl
from jax.experimental.pallas import tpu as pltpu
from jax.experimental.pallas import tpu_sc as plsc

N = 10000
E = 320000
IN = 128
HID = 128
CLS = 40

NC = 2
NS = 16
NW = NC * NS
EW = E // NW
CHUNK = 80
NCHUNK = EW // CHUNK
RPT = N // NS
ZROWS = 25


def _fill(ref, rows, width, value):
    vec = jnp.full((16,), value, dtype=jnp.float32)
    per_row = width // 16

    def body(t, _):
        r = t // per_row
        col = (t % per_row) * 16
        ref[r, pl.ds(col, 16)] = vec
        return _

    lax.fori_loop(0, rows * per_row, body, None)


def _sc_mesh():
    return plsc.VectorSubcoreMesh(core_axis_name="c", subcore_axis_name="s",
                                  num_cores=NC, num_subcores=NS)


def _sc_params():
    return pltpu.CompilerParams(use_tc_tiling_on_sc=False)


def _make_sc_agg(feat_width, interpret=False):
    scratch = [
        pltpu.VMEM((NCHUNK, CHUNK), jnp.int32),
        pltpu.VMEM((NCHUNK, CHUNK), jnp.int32),
        pltpu.VMEM((CHUNK, feat_width), jnp.float32),
        pltpu.VMEM((CHUNK, feat_width), jnp.float32),
        pltpu.VMEM((ZROWS, feat_width), jnp.float32),
        pltpu.VMEM_SHARED((N, feat_width), jnp.float32),
        pltpu.SemaphoreType.DMA,
        pltpu.SemaphoreType.DMA,
    ]

    @functools.partial(
        pl.kernel,
        out_type=jax.ShapeDtypeStruct((NC * N, feat_width), jnp.float32),
        mesh=_sc_mesh(),
        scratch_types=scratch,
        compiler_params=_sc_params(),
        interpret=interpret,
    )
    def sc_agg(p_hbm, src_hbm, dst_hbm, out_hbm, src_v, dst_v, rows0, rows1,
               zbuf, acc, sem0, sem1):
        c = lax.axis_index("c")
        s = lax.axis_index("s")
        w = c * NS + s
        rbase = s * RPT

        pltpu.sync_copy(src_hbm.at[pl.ds(w * NCHUNK, NCHUNK)], src_v)
        pltpu.sync_copy(dst_hbm.at[pl.ds(w * NCHUNK, NCHUNK)], dst_v)

        _fill(zbuf, ZROWS, feat_width, 0.0)

        def zero_body(i, _):
            pltpu.sync_copy(zbuf, acc.at[pl.ds(rbase + i * ZROWS, ZROWS)])
            return _

        lax.fori_loop(0, RPT // ZROWS, zero_body, None)
        plsc.subcore_barrier()

        def gather(j, buf, sem):
            pltpu.async_copy(p_hbm.at[src_v.at[j]], buf, sem)

        def gwait(buf, sem):
            pltpu.make_async_copy(p_hbm.at[src_v.at[0]], buf, sem).wait()

        def scat(j, buf):
            pltpu.sync_copy(buf, acc.at[dst_v.at[j]], add=True)

        gather(0, rows0, sem0)

        def body(i, _):
            j = 2 * i
            gather(j + 1, rows1, sem1)
            gwait(rows0, sem0)
            scat(j, rows0)
            gather(j + 2, rows0, sem0)
            gwait(rows1, sem1)
            scat(j + 1, rows1)
            return _

        lax.fori_loop(0, (NCHUNK - 1) // 2, body, None)
        gwait(rows0, sem0)
        scat(NCHUNK - 1, rows0)
        plsc.subcore_barrier()

        obase = c * N + rbase
        pltpu.sync_copy(acc.at[pl.ds(rbase, RPT)], out_hbm.at[pl.ds(obase, RPT)])

    return sc_agg


def _make_sc_deg(interpret=False):
    scratch = [
        pltpu.VMEM((NCHUNK, CHUNK), jnp.int32),
        pltpu.VMEM((CHUNK, 16), jnp.float32),
        pltpu.VMEM((ZROWS, 16), jnp.float32),
        pltpu.VMEM_SHARED((N, 16), jnp.float32),
        pltpu.SemaphoreType.DMA,
    ]

    @functools.partial(
        pl.kernel,
        out_type=jax.ShapeDtypeStruct((NC * N, 16), jnp.float32),
        mesh=_sc_mesh(),
        scratch_types=scratch,
        compiler_params=_sc_params(),
        interpret=interpret,
    )
    def sc_deg(dst_hbm, deg_hbm, dst_v, ones_v, zbuf16, dacc, sem):
        c = lax.axis_index("c")
        s = lax.axis_index("s")
        w = c * NS + s
        rbase = s * RPT

        pltpu.sync_copy(dst_hbm.at[pl.ds(w * NCHUNK, NCHUNK)], dst_v)
        _fill(ones_v, CHUNK, 16, 1.0)
        _fill(zbuf16, ZROWS, 16, 0.0)

        def zero_body(i, _):
            pltpu.sync_copy(zbuf16, dacc.at[pl.ds(rbase + i * ZROWS, ZROWS)])
            return _

        lax.fori_loop(0, RPT // ZROWS, zero_body, None)
        plsc.subcore_barrier()

        GRP = 5

        def body(i, _):
            def fire(g, _):
                pltpu.async_copy(ones_v, dacc.at[dst_v.at[i * GRP + g]], sem,
                                 add=True)
                return _

            lax.fori_loop(0, GRP, fire, None)

            def drain(g, _):
                pltpu.make_async_copy(ones_v, dacc.at[dst_v.at[0]], sem).wait()
                return _

            lax.fori_loop(0, GRP, drain, None)
            return _

        lax.fori_loop(0, NCHUNK // GRP, body, None)
        plsc.subcore_barrier()

        obase = c * N + rbase
        pltpu.sync_copy(dacc.at[pl.ds(rbase, RPT)], deg_hbm.at[pl.ds(obase, RPT)])

    return sc_deg


_make_sc_agg = functools.lru_cache(maxsize=None)(_make_sc_agg)
_make_sc_deg = functools.lru_cache(maxsize=None)(_make_sc_deg)

BN = 2000


def _row_spec(width):
    return pl.BlockSpec((BN, width), lambda i: (i, 0))


def _full_spec(shape):
    return pl.BlockSpec(shape, lambda i: tuple(0 for _ in shape))


def _mm_first_body(x_ref, wn_ref, ws_ref, b_ref, p_ref, s_ref):
    x = x_ref[...]
    p_ref[...] = jnp.dot(x, wn_ref[...], preferred_element_type=jnp.float32)
    s_ref[...] = (jnp.dot(x, ws_ref[...], preferred_element_type=jnp.float32)
                  + b_ref[...])


def _mm_first(x, wn, ws, b, interpret=False):
    return pl.pallas_call(
        _mm_first_body,
        grid=(N // BN,),
        in_specs=[_row_spec(IN), _full_spec((IN, HID)), _full_spec((IN, HID)),
                  _full_spec((1, HID))],
        out_specs=[_row_spec(HID), _row_spec(HID)],
        out_shape=[jax.ShapeDtypeStruct((N, HID), jnp.float32),
                   jax.ShapeDtypeStruct((N, HID), jnp.float32)],
        interpret=interpret,
    )(x, wn, ws, b)


def _mm_mid_body(sp_ref, a0_ref, a1_ref, d0_ref, d1_ref, wn_ref, ws_ref,
                 b_ref, p_ref, s_ref, inv_ref):
    deg = d0_ref[...][:, :1] + d1_ref[...][:, :1]
    inv = 1.0 / jnp.maximum(deg, 1.0)
    h = jnp.maximum(sp_ref[...] + (a0_ref[...] + a1_ref[...]) * inv, 0.0)
    p_ref[...] = jnp.dot(h, wn_ref[...], preferred_element_type=jnp.float32)
    s_ref[...] = (jnp.dot(h, ws_ref[...], preferred_element_type=jnp.float32)
                  + b_ref[...])
    inv_ref[...] = jnp.broadcast_to(inv, (BN, 16))


def _mm_mid(s_prev, a0, a1, d0, d1, wn, ws, b, interpret=False):
    return pl.pallas_call(
        _mm_mid_body,
        grid=(N // BN,),
        in_specs=[_row_spec(HID), _row_spec(HID), _row_spec(HID),
                  _row_spec(16), _row_spec(16),
                  _full_spec((HID, HID)), _full_spec((HID, HID)),
                  _full_spec((1, HID))],
        out_specs=[_row_spec(HID), _row_spec(HID), _row_spec(16)],
        out_shape=[jax.ShapeDtypeStruct((N, HID), jnp.float32),
                   jax.ShapeDtypeStruct((N, HID), jnp.float32),
                   jax.ShapeDtypeStruct((N, 16), jnp.float32)],
        interpret=interpret,
    )(s_prev, a0, a1, d0, d1, wn, ws, b)


def _mm_last_body(sp_ref, a0_ref, a1_ref, inv_ref, wn_ref, ws_ref, b_ref,
                  p_ref, s_ref):
    inv = inv_ref[...][:, :1]
    h = jnp.maximum(sp_ref[...] + (a0_ref[...] + a1_ref[...]) * inv, 0.0)
    p_ref[...] = jnp.dot(h, wn_ref[...], preferred_element_type=jnp.float32)
    s_ref[...] = (jnp.dot(h, ws_ref[...], preferred_element_type=jnp.float32)
                  + b_ref[...])


def _mm_last(s_prev, a0, a1, inv, wn_pad, ws, b, interpret=False):
    return pl.pallas_call(
        _mm_last_body,
        grid=(N // BN,),
        in_specs=[_row_spec(HID), _row_spec(HID), _row_spec(HID),
                  _row_spec(16),
                  _full_spec((HID, 64)), _full_spec((HID, CLS)),
                  _full_spec((1, CLS))],
        out_specs=[_row_spec(64), _row_spec(CLS)],
        out_shape=[jax.ShapeDtypeStruct((N, 64), jnp.float32),
                   jax.ShapeDtypeStruct((N, CLS), jnp.float32)],
        interpret=interpret,
    )(s_prev, a0, a1, inv, wn_pad, ws, b)


def _final_body(s_ref, a0_ref, a1_ref, inv_ref, o_ref):
    agg = a0_ref[...][:, :CLS] + a1_ref[...][:, :CLS]
    o_ref[...] = s_ref[...] + agg * inv_ref[...][:, :1]


def _final(s2, a0, a1, inv, interpret=False):
    return pl.pallas_call(
        _final_body,
        grid=(N // BN,),
        in_specs=[_row_spec(CLS), _row_spec(64), _row_spec(64),
                  _row_spec(16)],
        out_specs=_row_spec(CLS),
        out_shape=jax.ShapeDtypeStruct((N, CLS), jnp.float32),
        interpret=interpret,
    )(s2, a0, a1, inv)


def kernel(features, edge_index, W_self_0, W_neigh_0, b_0, W_self_1,
           W_neigh_1, b_1, W_self_2, W_neigh_2, b_2):
    src = edge_index[0].reshape(E // CHUNK, CHUNK)
    dst = edge_index[1].reshape(E // CHUNK, CHUNK)

    p0, s0 = _mm_first(features, W_neigh_0, W_self_0, b_0.reshape(1, HID))
    deg = _make_sc_deg()(dst)
    agg0 = _make_sc_agg(HID)(p0, src, dst)
    a0, a1 = agg0[:N], agg0[N:]
    d0, d1 = deg[:N], deg[N:]

    p1, s1, inv = _mm_mid(s0, a0, a1, d0, d1, W_neigh_1, W_self_1,
                          b_1.reshape(1, HID))
    agg1 = _make_sc_agg(HID)(p1, src, dst)

    wn2_pad = jnp.pad(W_neigh_2, ((0, 0), (0, 64 - CLS)))
    p2, s2 = _mm_last(s1, agg1[:N], agg1[N:], inv, wn2_pad, W_self_2,
                      b_2.reshape(1, CLS))
    agg2 = _make_sc_agg(64)(p2, src, dst)

    return _final(s2, agg2[:N], agg2[N:], inv)
